# baseline (speedup 1.0000x reference)
import functools

import jax
import jax.numpy as jnp
from jax import lax
from jax.experimental import pallas as pl
from jax.experimental.pallas import tpu as pltpu
from jax.experimental.pallas import tpu_sc as plsc

N = 10000
E = 320000
D0 = 128
D1 = 256
NC = 2
NS = 16
NW = NC * NS
EPT = E // NW
BS = 128
NB = 80
CH = 40
NCH = NB // CH
EPT_PAD = NB * BS
N_PAD = 10240
ROWS_PT = N_PAD // NS
ZR = 32
DW = 128

_MESH = plsc.VectorSubcoreMesh(core_axis_name="c", subcore_axis_name="s")


@functools.partial(
    pl.kernel,
    out_type=jax.ShapeDtypeStruct((NC, N_PAD, DW), jnp.float32),
    mesh=_MESH,
    scratch_types=[
        pltpu.VMEM((NB, BS), jnp.int32),
        pltpu.VMEM((BS, DW), jnp.float32),
        pltpu.VMEM_SHARED((N_PAD, DW), jnp.float32),
    ],
)
def _sc_degree(dst_hbm, ones_hbm, z_hbm, out_hbm, dst2,
               ones_v, acc_sh):
    cid = lax.axis_index("c")
    sid = lax.axis_index("s")
    wid = sid * NC + cid
    pltpu.sync_copy(ones_hbm, ones_v)

    def zero(i, carry):
        pltpu.sync_copy(z_hbm, acc_sh.at[pl.ds(sid * ROWS_PT + i * ZR, ZR)])
        return carry

    lax.fori_loop(0, ROWS_PT // ZR, zero, 0)
    plsc.subcore_barrier()

    pltpu.sync_copy(dst_hbm.at[wid], dst2)

    def body(j, carry):
        pltpu.sync_copy(ones_v, acc_sh.at[dst2.at[j]], add=True)
        return carry

    lax.fori_loop(0, NB, body, 0)
    plsc.subcore_barrier()
    pltpu.sync_copy(acc_sh.at[pl.ds(sid * ROWS_PT, ROWS_PT)],
                    out_hbm.at[cid, pl.ds(sid * ROWS_PT, ROWS_PT)])


@functools.partial(
    pl.kernel,
    out_type=jax.ShapeDtypeStruct((NC, N_PAD, D0), jnp.float32),
    mesh=_MESH,
    scratch_types=[
        pltpu.VMEM((NB, BS), jnp.int32),
        pltpu.VMEM((NB, BS), jnp.int32),
        pltpu.VMEM((BS, D0), jnp.float32),
        pltpu.VMEM_SHARED((N_PAD, D0), jnp.float32),
        pltpu.SemaphoreType.DMA,
    ],
)
def _sc_aggregate(table_hbm, src_hbm, dst_hbm, z_hbm, out_hbm, src2, dst2,
                  rows_v, acc_sh, sem):
    cid = lax.axis_index("c")
    sid = lax.axis_index("s")
    wid = sid * NC + cid

    def zero(i, carry):
        pltpu.sync_copy(z_hbm, acc_sh.at[pl.ds(sid * ROWS_PT + i * ZR, ZR)])
        return carry

    lax.fori_loop(0, ROWS_PT // ZR, zero, 0)
    plsc.subcore_barrier()

    pltpu.sync_copy(src_hbm.at[wid], src2)
    pltpu.sync_copy(dst_hbm.at[wid], dst2)

    def body(j, carry):
        pltpu.async_copy(table_hbm.at[src2.at[j]], rows_v, sem).wait()
        pltpu.sync_copy(rows_v, acc_sh.at[dst2.at[j]], add=True)
        return carry

    lax.fori_loop(0, NB, body, 0)
    plsc.subcore_barrier()
    pltpu.sync_copy(acc_sh.at[pl.ds(sid * ROWS_PT, ROWS_PT)],
                    out_hbm.at[cid, pl.ds(sid * ROWS_PT, ROWS_PT)])


def _tc_prep(x, degp_t):
    def body(x_ref, dp_ref, xt_ref, dinv_ref):
        deg = jnp.sum(dp_ref[...], axis=1, keepdims=True) + 1.0
        dinv = lax.rsqrt(deg)
        dinv_ref[...] = dinv
        xt_ref[...] = x_ref[...] * dinv

    return pl.pallas_call(
        body,
        out_shape=[
            jax.ShapeDtypeStruct((N, D0), jnp.float32),
            jax.ShapeDtypeStruct((N, 1), jnp.float32),
        ],
    )(x, degp_t)


def _graph_norm_relu(h, gamma, beta, alpha):
    mean = jnp.mean(h, axis=0, keepdims=True)
    o = h - alpha * mean
    var = jnp.mean(o * o, axis=0, keepdims=True)
    return jnp.maximum(gamma * o / jnp.sqrt(var + 1e-5) + beta, 0.0)


def _tc_layer0(s0a, s0b, xt, dinv, w0, b0, g0, be0, al0, w1):
    def body(s0a_ref, s0b_ref, xt_ref, dinv_ref, w0_ref, b0_ref, g0_ref,
             be0_ref, al0_ref, w1_ref, out_ref):
        dinv_v = dinv_ref[...]
        a0 = dinv_v * (s0a_ref[...] + s0b_ref[...] + xt_ref[...])
        h0 = jnp.dot(a0, w0_ref[...], preferred_element_type=jnp.float32)
        h0 = h0 + b0_ref[...]
        y0 = _graph_norm_relu(h0, g0_ref[...], be0_ref[...], al0_ref[...])
        h1 = jnp.dot(y0, w1_ref[...], preferred_element_type=jnp.float32)
        out_ref[...] = dinv_v * h1

    return pl.pallas_call(
        body,
        out_shape=jax.ShapeDtypeStruct((N, D0), jnp.float32),
    )(s0a, s0b, xt, dinv, w0, b0, g0, be0, al0, w1)


def _tc_layer1(s1a, s1b, ht1, dinv, b1, g1, be1, al1):
    def body(s1a_ref, s1b_ref, ht1_ref, dinv_ref, b1_ref, g1_ref, be1_ref,
             al1_ref, out_ref):
        a1 = dinv_ref[...] * (s1a_ref[...] + s1b_ref[...] + ht1_ref[...])
        a1 = a1 + b1_ref[...]
        out_ref[...] = _graph_norm_relu(a1, g1_ref[...], be1_ref[...],
                                        al1_ref[...])

    return pl.pallas_call(
        body,
        out_shape=jax.ShapeDtypeStruct((N, D0), jnp.float32),
    )(s1a, s1b, ht1, dinv, b1, g1, be1, al1)


def kernel(x, edge_index, W0, b0, gamma0, beta0, alpha0,
           W1, b1, gamma1, beta1, alpha1):
    src = edge_index[0]
    dst = edge_index[1]

    pad = EPT_PAD - EPT
    src3 = jnp.concatenate(
        [src.reshape(NW, EPT), jnp.zeros((NW, pad), jnp.int32)],
        axis=1).reshape(NW, NB, BS)
    dst3 = jnp.concatenate(
        [dst.reshape(NW, EPT), jnp.full((NW, pad), N, jnp.int32)],
        axis=1).reshape(NW, NB, BS)
    z = jnp.zeros((ZR, D0), jnp.float32)

    degp = _sc_degree(dst3, jnp.ones((BS, DW), jnp.float32), z)
    xt, dinv = _tc_prep(x, degp[:, :N, 0].T)

    s0 = _sc_aggregate(xt, src3, dst3, z)[:, :N]
    ht1 = _tc_layer0(
        s0[0], s0[1], xt, dinv, W0, b0.reshape(1, D1), gamma0.reshape(1, D1),
        beta0.reshape(1, D1), alpha0.reshape(1, D1), W1)

    s1 = _sc_aggregate(ht1, src3, dst3, z)[:, :N]
    return _tc_layer1(
        s1[0], s1[1], ht1, dinv, b1.reshape(1, D0), gamma1.reshape(1, D0),
        beta1.reshape(1, D0), alpha1.reshape(1, D0))


# speedup vs baseline: 1.8719x; 1.8719x over previous
import functools

import jax
import jax.numpy as jnp
from jax import lax
from jax.experimental import pallas as pl
from jax.experimental.pallas import tpu as pltpu
from jax.experimental.pallas import tpu_sc as plsc

N = 10000
E = 320000
D0 = 128
D1 = 256
NC = 2
NS = 16
NW = NC * NS
EPT = E // NW
BS = 80
NB = EPT // BS
N_PAD = 10240
ROWS_PT = N_PAD // NS
DW = 128

_MESH = plsc.VectorSubcoreMesh(core_axis_name="c", subcore_axis_name="s")


@functools.partial(
    pl.kernel,
    out_type=jax.ShapeDtypeStruct((NC, N_PAD, DW), jnp.float32),
    mesh=_MESH,
    scratch_types=[
        pltpu.VMEM((NB, BS), jnp.int32),
        pltpu.VMEM((BS, DW), jnp.float32),
        pltpu.VMEM_SHARED((N_PAD, DW), jnp.float32),
    ],
)
def _sc_degree(dst_hbm, ones_hbm, z_hbm, out_hbm, dst_v, ones_v, acc_sh):
    cid = lax.axis_index("c")
    sid = lax.axis_index("s")
    wid = sid * NC + cid
    pltpu.sync_copy(dst_hbm.at[wid], dst_v)
    pltpu.sync_copy(ones_hbm, ones_v)
    pltpu.sync_copy(z_hbm, acc_sh.at[pl.ds(sid * ROWS_PT, ROWS_PT)])
    plsc.subcore_barrier()

    def body(j, carry):
        pltpu.sync_copy(ones_v, acc_sh.at[dst_v.at[j]], add=True)
        return carry

    lax.fori_loop(0, NB, body, 0)
    plsc.subcore_barrier()
    pltpu.sync_copy(acc_sh.at[pl.ds(sid * ROWS_PT, ROWS_PT)],
                    out_hbm.at[cid, pl.ds(sid * ROWS_PT, ROWS_PT)])


@functools.partial(
    pl.kernel,
    out_type=jax.ShapeDtypeStruct((NC, N_PAD, D0), jnp.float32),
    mesh=_MESH,
    scratch_types=[
        pltpu.VMEM((NB, BS), jnp.int32),
        pltpu.VMEM((NB, BS), jnp.int32),
        pltpu.VMEM((BS, D0), jnp.float32),
        pltpu.VMEM_SHARED((N_PAD, D0), jnp.float32),
        pltpu.SemaphoreType.DMA,
    ],
)
def _sc_aggregate(table_hbm, src_hbm, dst_hbm, z_hbm, out_hbm,
                  src_v, dst_v, rows_v, acc_sh, sem):
    cid = lax.axis_index("c")
    sid = lax.axis_index("s")
    wid = sid * NC + cid
    pltpu.sync_copy(src_hbm.at[wid], src_v)
    pltpu.sync_copy(dst_hbm.at[wid], dst_v)
    pltpu.sync_copy(z_hbm, acc_sh.at[pl.ds(sid * ROWS_PT, ROWS_PT)])
    plsc.subcore_barrier()

    def body(j, carry):
        pltpu.async_copy(table_hbm.at[src_v.at[j]], rows_v, sem).wait()
        pltpu.sync_copy(rows_v, acc_sh.at[dst_v.at[j]], add=True)
        return carry

    lax.fori_loop(0, NB, body, 0)
    plsc.subcore_barrier()
    pltpu.sync_copy(acc_sh.at[pl.ds(sid * ROWS_PT, ROWS_PT)],
                    out_hbm.at[cid, pl.ds(sid * ROWS_PT, ROWS_PT)])


def _tc_prep(x, degp_t):
    def body(x_ref, dp_ref, xt_ref, dinv_ref):
        deg = jnp.sum(dp_ref[...], axis=1, keepdims=True) + 1.0
        dinv = lax.rsqrt(deg)
        dinv_ref[...] = dinv
        xt_ref[...] = x_ref[...] * dinv

    return pl.pallas_call(
        body,
        out_shape=[
            jax.ShapeDtypeStruct((N, D0), jnp.float32),
            jax.ShapeDtypeStruct((N, 1), jnp.float32),
        ],
    )(x, degp_t)


def _graph_norm_relu(h, gamma, beta, alpha):
    mean = jnp.mean(h, axis=0, keepdims=True)
    o = h - alpha * mean
    var = jnp.mean(o * o, axis=0, keepdims=True)
    return jnp.maximum(gamma * o / jnp.sqrt(var + 1e-5) + beta, 0.0)


def _tc_layer0(s0a, s0b, xt, dinv, w0, b0, g0, be0, al0, w1):
    def body(s0a_ref, s0b_ref, xt_ref, dinv_ref, w0_ref, b0_ref, g0_ref,
             be0_ref, al0_ref, w1_ref, out_ref):
        dinv_v = dinv_ref[...]
        a0 = dinv_v * (s0a_ref[...] + s0b_ref[...] + xt_ref[...])
        h0 = jnp.dot(a0, w0_ref[...], preferred_element_type=jnp.float32)
        h0 = h0 + b0_ref[...]
        y0 = _graph_norm_relu(h0, g0_ref[...], be0_ref[...], al0_ref[...])
        h1 = jnp.dot(y0, w1_ref[...], preferred_element_type=jnp.float32)
        out_ref[...] = dinv_v * h1

    return pl.pallas_call(
        body,
        out_shape=jax.ShapeDtypeStruct((N, D0), jnp.float32),
    )(s0a, s0b, xt, dinv, w0, b0, g0, be0, al0, w1)


def _tc_layer1(s1a, s1b, ht1, dinv, b1, g1, be1, al1):
    def body(s1a_ref, s1b_ref, ht1_ref, dinv_ref, b1_ref, g1_ref, be1_ref,
             al1_ref, out_ref):
        a1 = dinv_ref[...] * (s1a_ref[...] + s1b_ref[...] + ht1_ref[...])
        a1 = a1 + b1_ref[...]
        out_ref[...] = _graph_norm_relu(a1, g1_ref[...], be1_ref[...],
                                        al1_ref[...])

    return pl.pallas_call(
        body,
        out_shape=jax.ShapeDtypeStruct((N, D0), jnp.float32),
    )(s1a, s1b, ht1, dinv, b1, g1, be1, al1)


def kernel(x, edge_index, W0, b0, gamma0, beta0, alpha0,
           W1, b1, gamma1, beta1, alpha1):
    src = edge_index[0]
    dst = edge_index[1]

    src3 = src.reshape(NW, NB, BS)
    dst3 = dst.reshape(NW, NB, BS)
    z = jnp.zeros((ROWS_PT, D0), jnp.float32)

    degp = _sc_degree(dst3, jnp.ones((BS, DW), jnp.float32), z)
    xt, dinv = _tc_prep(x, degp[:, :N, 0].T)

    s0 = _sc_aggregate(xt, src3, dst3, z)[:, :N]
    ht1 = _tc_layer0(
        s0[0], s0[1], xt, dinv, W0, b0.reshape(1, D1), gamma0.reshape(1, D1),
        beta0.reshape(1, D1), alpha0.reshape(1, D1), W1)

    s1 = _sc_aggregate(ht1, src3, dst3, z)[:, :N]
    return _tc_layer1(
        s1[0], s1[1], ht1, dinv, b1.reshape(1, D0), gamma1.reshape(1, D0),
        beta1.reshape(1, D0), alpha1.reshape(1, D0))
